# baseline (speedup 1.0000x reference)
import functools

import jax
import jax.numpy as jnp
from jax import lax
from jax.experimental import pallas as pl
from jax.experimental.pallas import tpu as pltpu
from jax.experimental.pallas import tpu_sc as plsc

NC = 2
NS = 16
CHUNK = 128
GIDX = 8


def _sc_mesh():
    return plsc.VectorSubcoreMesh(core_axis_name="c", subcore_axis_name="s")


@functools.lru_cache(maxsize=None)
def _make_sc_pass(n_pad, c_pad, d, with_gather, c0_frac16=8):
    rows_per_tile = n_pad // NS
    quantum = NS * GIDX
    c0 = (c_pad * c0_frac16 // 16) // quantum * quantum
    core_chunks = (c0, c_pad - c0)
    core_starts = (0, c0)

    hops = []
    off = 0
    while off < rows_per_tile:
        sz = min(CHUNK, rows_per_tile - off)
        hops.append((off, sz))
        off += sz

    out_type = [jax.ShapeDtypeStruct((NC, n_pad, d), jnp.float32)]
    scratch = [
        pltpu.VMEM_SHARED((n_pad, d), jnp.float32),
        pltpu.VMEM((GIDX, CHUNK), jnp.int32),
        pltpu.VMEM((GIDX, CHUNK), jnp.int32),
        pltpu.VMEM((CHUNK, d), jnp.float32),
        pltpu.VMEM((CHUNK, d), jnp.float32),
        pltpu.SemaphoreType.DMA,
        pltpu.SemaphoreType.DMA,
    ]

    def body(ei_ref, h_ref, zacc_ref, ones_ref, acc_out, acc_sh, idx_s,
             idx_d, rows_a, rows_b, sem_a, sem_b):
        cid = lax.axis_index("c")
        sid = lax.axis_index("s")
        r0 = sid * rows_per_tile

        pltpu.sync_copy(zacc_ref.at[pl.ds(0, CHUNK)], rows_a)
        for off, sz in hops:
            pltpu.sync_copy(rows_a.at[pl.ds(0, sz)],
                            acc_sh.at[pl.ds(r0 + off, sz)])

        if not with_gather:
            pltpu.sync_copy(ones_ref, rows_a)
            pltpu.sync_copy(ones_ref, rows_b)

        plsc.subcore_barrier()

        bufs = (rows_a, rows_b)
        sems = (sem_a, sem_b)

        def make_group_body(start):
            def group_body(g, carry):
                g0 = start + g * GIDX
                pltpu.sync_copy(ei_ref.at[0, pl.ds(g0, GIDX)], idx_s)
                pltpu.sync_copy(ei_ref.at[1, pl.ds(g0, GIDX)], idx_d)

                if with_gather:
                    ops = []
                    for i in range(GIDX):
                        ops.append((idx_s.at[i], idx_d.at[i]))
                        ops.append((idx_d.at[i], idx_s.at[i]))
                    pend = pltpu.async_copy(h_ref.at[ops[0][0]], bufs[0],
                                            sems[0])
                    for j, (gsrc, sdst) in enumerate(ops):
                        buf = bufs[j % 2]
                        nxt = None
                        if j + 1 < len(ops):
                            nxt = pltpu.async_copy(h_ref.at[ops[j + 1][0]],
                                                   bufs[(j + 1) % 2],
                                                   sems[(j + 1) % 2])
                        pend.wait()
                        pltpu.sync_copy(buf, acc_sh.at[sdst], add=True)
                        pend = nxt
                else:
                    for i in range(GIDX):
                        pltpu.sync_copy(rows_a, acc_sh.at[idx_d.at[i]],
                                        add=True)
                        pltpu.sync_copy(rows_b, acc_sh.at[idx_s.at[i]],
                                        add=True)
                return carry
            return group_body

        for core in range(NC):
            if core_chunks[core] == 0:
                continue
            cpt = core_chunks[core] // NS
            body_fn = make_group_body(core_starts[core] + sid * cpt)

            @pl.when(cid == core)
            def _(body_fn=body_fn, cpt=cpt):
                lax.fori_loop(0, cpt // GIDX, body_fn, 0)

        plsc.subcore_barrier()

        for off, sz in hops:
            pltpu.sync_copy(acc_sh.at[pl.ds(r0 + off, sz)],
                            rows_a.at[pl.ds(0, sz)])
            pltpu.sync_copy(rows_a.at[pl.ds(0, sz)],
                            acc_out.at[cid, pl.ds(r0 + off, sz)])

    return pl.kernel(body, out_type=out_type, mesh=_sc_mesh(),
                     scratch_types=scratch)


def _make_tc_layer(n, d, blk):
    def body(h_ref, acc_ref, deg_ref, w_ref, b_ref, o_ref):
        deg = deg_ref[0][:, 0:1] + deg_ref[1][:, 0:1]
        scale = 1.0 / jnp.maximum(deg, 1.0)
        agg = (acc_ref[0] + acc_ref[1]) * scale
        x = h_ref[...] + agg
        y = jnp.dot(x, w_ref[...], preferred_element_type=jnp.float32)
        o_ref[...] = jnp.maximum(y + b_ref[...], 0.0)

    return pl.pallas_call(
        body,
        grid=(n // blk,),
        in_specs=[
            pl.BlockSpec((blk, d), lambda i: (i, 0)),
            pl.BlockSpec((NC, blk, d), lambda i: (0, i, 0)),
            pl.BlockSpec((NC, blk, d), lambda i: (0, i, 0)),
            pl.BlockSpec((d, d), lambda i: (0, 0)),
            pl.BlockSpec((1, d), lambda i: (0, 0)),
        ],
        out_specs=pl.BlockSpec((blk, d), lambda i: (i, 0)),
        out_shape=jax.ShapeDtypeStruct((n, d), jnp.float32),
    )


def _pad_rows(x, n_pad):
    n = x.shape[0]
    return jnp.concatenate(
        [x, jnp.zeros((n_pad - n, x.shape[1]), jnp.float32)], axis=0)


def kernel(h, edge_index, W0, b0, W1, b1):
    n, d = h.shape
    e = edge_index.shape[1]

    n_pad = (n // 128 + 1) * 128
    cq = CHUNK * NC * NS * GIDX
    e_pad = -(-e // cq) * cq
    c_pad = e_pad // CHUNK

    ei_pad = jnp.concatenate(
        [edge_index, jnp.full((2, e_pad - e), n, jnp.int32)], axis=1)
    ei_r = ei_pad.reshape(2, c_pad, CHUNK)

    zacc = jnp.zeros((n_pad // NS, d), jnp.float32)
    ones = jnp.ones((CHUNK, d), jnp.float32)

    agg_pass = _make_sc_pass(n_pad, c_pad, d, True, 4)
    deg_pass = _make_sc_pass(n_pad, c_pad, d, False, 8)
    tc_layer = _make_tc_layer(n, d, 400)

    h_pad = _pad_rows(h, n_pad)
    (deg,) = deg_pass(ei_r, h_pad, zacc, ones)
    (acc1,) = agg_pass(ei_r, h_pad, zacc, ones)
    h1 = tc_layer(h, acc1, deg, W0, b0.reshape(1, d))

    (acc2,) = agg_pass(ei_r, _pad_rows(h1, n_pad), zacc, ones)
    out = tc_layer(h1, acc2, deg, W1, b1.reshape(1, d))
    return out


# speedup vs baseline: 1.1049x; 1.1049x over previous
import functools

import jax
import jax.numpy as jnp
from jax import lax
from jax.experimental import pallas as pl
from jax.experimental.pallas import tpu as pltpu
from jax.experimental.pallas import tpu_sc as plsc

NC = 2
NS = 16
CHUNK = 128
GIDX = 8


def _sc_mesh():
    return plsc.VectorSubcoreMesh(core_axis_name="c", subcore_axis_name="s")


@functools.lru_cache(maxsize=None)
def _make_sc_pass(n_pad, c_pad, d, with_gather, c0_frac16=8):
    rows_per_tile = n_pad // NS
    quantum = NS * GIDX
    c0 = (c_pad * c0_frac16 // 16) // quantum * quantum
    core_chunks = (c0, c_pad - c0)
    core_starts = (0, c0)

    hops = []
    off = 0
    while off < rows_per_tile:
        sz = min(CHUNK, rows_per_tile - off)
        hops.append((off, sz))
        off += sz

    out_type = [jax.ShapeDtypeStruct((NC, n_pad, d), jnp.float32)]
    scratch = [
        pltpu.VMEM_SHARED((n_pad, d), jnp.float32),
        pltpu.VMEM((GIDX, CHUNK), jnp.int32),
        pltpu.VMEM((GIDX, CHUNK), jnp.int32),
        pltpu.VMEM((CHUNK, d), jnp.float32),
        pltpu.VMEM((CHUNK, d), jnp.float32),
        pltpu.SemaphoreType.DMA,
        pltpu.SemaphoreType.DMA,
    ]

    def body(ei_ref, h_ref, zacc_ref, ones_ref, acc_out, acc_sh, idx_s,
             idx_d, rows_a, rows_b, sem_a, sem_b):
        cid = lax.axis_index("c")
        sid = lax.axis_index("s")
        r0 = sid * rows_per_tile

        pltpu.sync_copy(zacc_ref.at[pl.ds(0, CHUNK)], rows_a)
        for off, sz in hops:
            pltpu.sync_copy(rows_a.at[pl.ds(0, sz)],
                            acc_sh.at[pl.ds(r0 + off, sz)])

        if not with_gather:
            pltpu.sync_copy(ones_ref, rows_a)
            pltpu.sync_copy(ones_ref, rows_b)

        plsc.subcore_barrier()

        bufs = (rows_a, rows_b)
        sems = (sem_a, sem_b)

        def make_group_body(start):
            def group_body(g, carry):
                g0 = start + g * GIDX
                pltpu.sync_copy(ei_ref.at[0, pl.ds(g0, GIDX)], idx_s)
                pltpu.sync_copy(ei_ref.at[1, pl.ds(g0, GIDX)], idx_d)

                if with_gather:
                    ops = []
                    for i in range(GIDX):
                        ops.append((idx_s.at[i], idx_d.at[i]))
                        ops.append((idx_d.at[i], idx_s.at[i]))
                    pend = pltpu.async_copy(h_ref.at[ops[0][0]], bufs[0],
                                            sems[0])
                    for j, (gsrc, sdst) in enumerate(ops):
                        buf = bufs[j % 2]
                        nxt = None
                        if j + 1 < len(ops):
                            nxt = pltpu.async_copy(h_ref.at[ops[j + 1][0]],
                                                   bufs[(j + 1) % 2],
                                                   sems[(j + 1) % 2])
                        pend.wait()
                        pltpu.sync_copy(buf, acc_sh.at[sdst], add=True)
                        pend = nxt
                else:
                    for i in range(GIDX):
                        pltpu.sync_copy(rows_a, acc_sh.at[idx_d.at[i]],
                                        add=True)
                        pltpu.sync_copy(rows_b, acc_sh.at[idx_s.at[i]],
                                        add=True)
                return carry
            return group_body

        for core in range(NC):
            if core_chunks[core] == 0:
                continue
            cpt = core_chunks[core] // NS
            body_fn = make_group_body(core_starts[core] + sid * cpt)

            @pl.when(cid == core)
            def _(body_fn=body_fn, cpt=cpt):
                lax.fori_loop(0, cpt // GIDX, body_fn, 0)

        plsc.subcore_barrier()

        for off, sz in hops:
            pltpu.sync_copy(acc_sh.at[pl.ds(r0 + off, sz)],
                            rows_a.at[pl.ds(0, sz)])
            pltpu.sync_copy(rows_a.at[pl.ds(0, sz)],
                            acc_out.at[cid, pl.ds(r0 + off, sz)])

    return pl.kernel(body, out_type=out_type, mesh=_sc_mesh(),
                     scratch_types=scratch)


def _make_tc_layer(n, d, blk):
    def body(h_ref, acc_ref, deg_ref, w_ref, b_ref, o_ref):
        deg = deg_ref[0][:, 0:1] + deg_ref[1][:, 0:1]
        scale = 1.0 / jnp.maximum(deg, 1.0)
        agg = (acc_ref[0] + acc_ref[1]) * scale
        x = h_ref[...] + agg
        y = jnp.dot(x, w_ref[...], preferred_element_type=jnp.float32)
        o_ref[...] = jnp.maximum(y + b_ref[...], 0.0)

    return pl.pallas_call(
        body,
        grid=(n // blk,),
        in_specs=[
            pl.BlockSpec((blk, d), lambda i: (i, 0)),
            pl.BlockSpec((NC, blk, d), lambda i: (0, i, 0)),
            pl.BlockSpec((NC, blk, d), lambda i: (0, i, 0)),
            pl.BlockSpec((d, d), lambda i: (0, 0)),
            pl.BlockSpec((1, d), lambda i: (0, 0)),
        ],
        out_specs=pl.BlockSpec((blk, d), lambda i: (i, 0)),
        out_shape=jax.ShapeDtypeStruct((n, d), jnp.float32),
    )


def _pad_rows(x, n_pad):
    n = x.shape[0]
    return jnp.concatenate(
        [x, jnp.zeros((n_pad - n, x.shape[1]), jnp.float32)], axis=0)


def kernel(h, edge_index, W0, b0, W1, b1):
    n, d = h.shape
    e = edge_index.shape[1]

    n_pad = (n // 128 + 1) * 128
    cq = CHUNK * NC * NS * GIDX
    e_pad = -(-e // cq) * cq
    c_pad = e_pad // CHUNK

    ei_pad = jnp.concatenate(
        [edge_index, jnp.full((2, e_pad - e), n, jnp.int32)], axis=1)
    ei_r = ei_pad.reshape(2, c_pad, CHUNK)

    zacc = jnp.zeros((n_pad // NS, d), jnp.float32)
    ones = jnp.ones((CHUNK, d), jnp.float32)

    agg_pass = _make_sc_pass(n_pad, c_pad, d, True, 12)
    deg_pass = _make_sc_pass(n_pad, c_pad, d, False, 8)
    tc_layer = _make_tc_layer(n, d, 400)

    h_pad = _pad_rows(h, n_pad)
    (deg,) = deg_pass(ei_r, h_pad, zacc, ones)
    (acc1,) = agg_pass(ei_r, h_pad, zacc, ones)
    h1 = tc_layer(h, acc1, deg, W0, b0.reshape(1, d))

    (acc2,) = agg_pass(ei_r, _pad_rows(h1, n_pad), zacc, ones)
    out = tc_layer(h1, acc2, deg, W1, b1.reshape(1, d))
    return out


# speedup vs baseline: 2.7705x; 2.5076x over previous
import functools

import jax
import jax.numpy as jnp
from jax import lax
from jax.experimental import pallas as pl
from jax.experimental.pallas import tpu as pltpu
from jax.experimental.pallas import tpu_sc as plsc

NC = 2
NS = 16
CHUNK = 128
GIDX = 8


def _sc_mesh():
    return plsc.VectorSubcoreMesh(core_axis_name="c", subcore_axis_name="s")


@functools.lru_cache(maxsize=None)
def _make_sc_pass(n_pad, c_pad, d, with_gather, c0_frac16=8):
    rows_per_tile = n_pad // NS
    quantum = NS * GIDX
    c0 = (c_pad * c0_frac16 // 16) // quantum * quantum
    core_chunks = (c0, c_pad - c0)
    core_starts = (0, c0)

    hops = []
    off = 0
    while off < rows_per_tile:
        sz = min(CHUNK, rows_per_tile - off)
        hops.append((off, sz))
        off += sz

    out_type = [jax.ShapeDtypeStruct((NC, n_pad, d), jnp.float32)]
    scratch = [
        pltpu.VMEM_SHARED((n_pad, d), jnp.float32),
        pltpu.VMEM((GIDX, CHUNK), jnp.int32),
        pltpu.VMEM((GIDX, CHUNK), jnp.int32),
        pltpu.VMEM((CHUNK, d), jnp.float32),
        pltpu.VMEM((CHUNK, d), jnp.float32),
        pltpu.SemaphoreType.DMA,
        pltpu.SemaphoreType.DMA,
    ]

    def body(ei_ref, h_ref, zacc_ref, ones_ref, acc_out, acc_sh, idx_s,
             idx_d, rows_a, rows_b, sem_a, sem_b):
        cid = lax.axis_index("c")
        sid = lax.axis_index("s")
        r0 = sid * rows_per_tile

        pltpu.sync_copy(zacc_ref.at[pl.ds(0, CHUNK)], rows_a)
        for off, sz in hops:
            pltpu.sync_copy(rows_a.at[pl.ds(0, sz)],
                            acc_sh.at[pl.ds(r0 + off, sz)])

        if not with_gather:
            pltpu.sync_copy(ones_ref, rows_a)
            pltpu.sync_copy(ones_ref, rows_b)

        plsc.subcore_barrier()

        bufs = (rows_a, rows_b)
        sems = (sem_a, sem_b)

        def make_group_body(start):
            def group_body(g, carry):
                g0 = start + g * GIDX
                pltpu.sync_copy(ei_ref.at[0, pl.ds(g0, GIDX)], idx_s)
                pltpu.sync_copy(ei_ref.at[1, pl.ds(g0, GIDX)], idx_d)

                if with_gather:
                    ops = []
                    for i in range(GIDX):
                        ops.append((idx_s.at[i], idx_d.at[i]))
                        ops.append((idx_d.at[i], idx_s.at[i]))
                    pend = pltpu.async_copy(h_ref.at[ops[0][0]], bufs[0],
                                            sems[0])
                    for j, (gsrc, sdst) in enumerate(ops):
                        buf = bufs[j % 2]
                        nxt = None
                        if j + 1 < len(ops):
                            nxt = pltpu.async_copy(h_ref.at[ops[j + 1][0]],
                                                   bufs[(j + 1) % 2],
                                                   sems[(j + 1) % 2])
                        pend.wait()
                        pltpu.sync_copy(buf, acc_sh.at[sdst], add=True)
                        pend = nxt
                else:
                    for i in range(GIDX):
                        pltpu.sync_copy(rows_a, acc_sh.at[idx_d.at[i]],
                                        add=True)
                        pltpu.sync_copy(rows_b, acc_sh.at[idx_s.at[i]],
                                        add=True)
                return carry
            return group_body

        for core in range(NC):
            if core_chunks[core] == 0:
                continue
            cpt = core_chunks[core] // NS
            body_fn = make_group_body(core_starts[core] + sid * cpt)

            @pl.when(cid == core)
            def _(body_fn=body_fn, cpt=cpt):
                lax.fori_loop(0, cpt // GIDX, body_fn, 0)

        plsc.subcore_barrier()

        for off, sz in hops:
            pltpu.sync_copy(acc_sh.at[pl.ds(r0 + off, sz)],
                            rows_a.at[pl.ds(0, sz)])
            pltpu.sync_copy(rows_a.at[pl.ds(0, sz)],
                            acc_out.at[cid, pl.ds(r0 + off, sz)])

    return pl.kernel(body, out_type=out_type, mesh=_sc_mesh(),
                     scratch_types=scratch)


def _make_tc_layer(n, d, blk):
    def body(h_ref, acc_ref, deg_ref, w_ref, b_ref, o_ref):
        deg = deg_ref[0][:, 0:1] + deg_ref[1][:, 0:1]
        scale = 1.0 / jnp.maximum(deg, 1.0)
        agg = (acc_ref[0] + acc_ref[1]) * scale
        x = h_ref[...] + agg
        y = jnp.dot(x, w_ref[...], preferred_element_type=jnp.float32)
        o_ref[...] = jnp.maximum(y + b_ref[...], 0.0)

    return pl.pallas_call(
        body,
        grid=(n // blk,),
        in_specs=[
            pl.BlockSpec((blk, d), lambda i: (i, 0)),
            pl.BlockSpec((NC, blk, d), lambda i: (0, i, 0)),
            pl.BlockSpec((NC, blk, d), lambda i: (0, i, 0)),
            pl.BlockSpec((d, d), lambda i: (0, 0)),
            pl.BlockSpec((1, d), lambda i: (0, 0)),
        ],
        out_specs=pl.BlockSpec((blk, d), lambda i: (i, 0)),
        out_shape=jax.ShapeDtypeStruct((n, d), jnp.float32),
    )


def _pad_rows(x, n_pad):
    n = x.shape[0]
    return jnp.concatenate(
        [x, jnp.zeros((n_pad - n, x.shape[1]), jnp.float32)], axis=0)


def kernel(h, edge_index, W0, b0, W1, b1):
    n, d = h.shape
    e = edge_index.shape[1]

    n_pad = (n // 128 + 1) * 128
    cq = CHUNK * NC * NS * GIDX
    e_pad = -(-e // cq) * cq
    c_pad = e_pad // CHUNK

    pad_idx = n + jnp.arange(e_pad - e, dtype=jnp.int32) % (n_pad - n)
    ei_pad = jnp.concatenate(
        [edge_index, jnp.broadcast_to(pad_idx, (2, e_pad - e))], axis=1)
    ei_r = ei_pad.reshape(2, c_pad, CHUNK)

    zacc = jnp.zeros((n_pad // NS, d), jnp.float32)
    ones = jnp.ones((CHUNK, d), jnp.float32)

    agg_pass = _make_sc_pass(n_pad, c_pad, d, True, 8)
    deg_pass = _make_sc_pass(n_pad, c_pad, d, False, 8)
    tc_layer = _make_tc_layer(n, d, 400)

    h_pad = _pad_rows(h, n_pad)
    (deg,) = deg_pass(ei_r, h_pad, zacc, ones)
    (acc1,) = agg_pass(ei_r, h_pad, zacc, ones)
    h1 = tc_layer(h, acc1, deg, W0, b0.reshape(1, d))

    (acc2,) = agg_pass(ei_r, _pad_rows(h1, n_pad), zacc, ones)
    out = tc_layer(h1, acc2, deg, W1, b1.reshape(1, d))
    return out
